# Initial kernel scaffold; baseline (speedup 1.0000x reference)
import jax
import jax.numpy as jnp
from jax.experimental import pallas as pl


def kernel(query_x, query_edge_index, query_graph_idx, corpus_x, corpus_edge_index, corpus_graph_idx, batch_size, params):
    raise NotImplementedError("write your pallas kernel here")


# speedup vs baseline: 3.1414x; 3.1414x over previous
import functools

import jax
import jax.numpy as jnp
from jax import lax
from jax.experimental import pallas as pl
from jax.experimental.pallas import tpu as pltpu
from jax.experimental.pallas import tpu_sc as plsc

_N = 10000
_NP = 10240
_E = 320000
_B = 128
_D = 64
_IN = 128
_K = 80
_TILES = 16
_EPT = _E // _TILES
_CH = _EPT // _K
_RPT = _NP // _TILES
_R = 2048
_G = (2 * _NP) // _R


def _edge_agg_body(z_hbm, src_hbm, dst_hbm, zero_hbm, out_hbm,
                   src_v, dst_v, rows_v, agg_sh, sem):
    c = lax.axis_index("c")
    s = lax.axis_index("s")
    pltpu.sync_copy(zero_hbm, agg_sh.at[pl.ds(s * _RPT, _RPT)])
    plsc.subcore_barrier()
    ebase = c * _E + s * _EPT

    def body(j, carry):
        base = pl.multiple_of(ebase + j * _K, _K)
        pltpu.sync_copy(src_hbm.at[pl.ds(base, _K)], src_v)
        pltpu.sync_copy(dst_hbm.at[pl.ds(base, _K)], dst_v)
        pltpu.async_copy(z_hbm.at[src_v], rows_v, sem).wait()
        pltpu.sync_copy(rows_v, agg_sh.at[dst_v], add=True)
        return carry

    lax.fori_loop(0, _CH, body, 0)
    plsc.subcore_barrier()
    rbase = s * _RPT
    pltpu.sync_copy(agg_sh.at[pl.ds(rbase, _RPT)],
                    out_hbm.at[pl.ds(c * _NP + rbase, _RPT)])


@functools.cache
def _get_edge_agg():
    return pl.kernel(
        _edge_agg_body,
        out_type=jax.ShapeDtypeStruct((2 * _NP, _D), jnp.float32),
        mesh=plsc.VectorSubcoreMesh(core_axis_name="c", subcore_axis_name="s"),
        scratch_types=[
            pltpu.VMEM((_K,), jnp.int32),
            pltpu.VMEM((_K,), jnp.int32),
            pltpu.VMEM((_K, _D), jnp.float32),
            pltpu.VMEM_SHARED((_NP, _D), jnp.float32),
            pltpu.SemaphoreType.DMA,
        ],
        compiler_params=pltpu.CompilerParams(use_tc_tiling_on_sc=False),
    )


def _pre_body(x_ref, w_ref, o_ref):
    o_ref[...] = jnp.dot(x_ref[...], w_ref[...],
                         preferred_element_type=jnp.float32)


_pre = pl.pallas_call(
    _pre_body,
    grid=(_G,),
    in_specs=[
        pl.BlockSpec((_R, _IN), lambda i: (i, 0)),
        pl.BlockSpec((_IN, _D), lambda i: (0, 0)),
    ],
    out_specs=pl.BlockSpec((_R, _D), lambda i: (i, 0)),
    out_shape=jax.ShapeDtypeStruct((2 * _NP, _D), jnp.float32),
)


def _layer_body(has_next):
    def body(z_ref, a_ref, g_ref, b1_ref, w2_ref, b2_ref, bng_ref, bnb_ref,
             iw_ref, ib_ref, *rest):
        if has_next:
            w1n_ref, pooled_ref, znext_ref = rest
        else:
            (pooled_ref,) = rest
        i = pl.program_id(0)
        h = jnp.maximum(z_ref[...] + a_ref[...] + b1_ref[...], 0.0)
        h2 = jnp.dot(h, w2_ref[...], preferred_element_type=jnp.float32)
        h2 = h2 + b2_ref[...]
        x_new = jnp.maximum(h2 * bng_ref[...] + bnb_ref[...], 0.0)
        y = jnp.dot(x_new, iw_ref[...], preferred_element_type=jnp.float32)
        y = jnp.maximum(y + ib_ref[...], 0.0)
        g = g_ref[0, 0, :]
        oh = (g[None, :] == lax.broadcasted_iota(jnp.int32, (2 * _B, _R), 0))
        ps = jnp.dot(oh.astype(jnp.float32), y,
                     preferred_element_type=jnp.float32)

        @pl.when(i == 0)
        def _():
            pooled_ref[...] = ps

        @pl.when(i > 0)
        def _():
            pooled_ref[...] += ps

        if has_next:
            znext_ref[...] = jnp.dot(x_new, w1n_ref[...],
                                     preferred_element_type=jnp.float32)

    return body


def _make_layer(has_next):
    body = _layer_body(has_next)
    vec = pl.BlockSpec((1, _D), lambda i: (0, 0))
    mat = pl.BlockSpec((_D, _D), lambda i: (0, 0))
    rowblk = pl.BlockSpec((_R, _D), lambda i: (i, 0))
    in_specs = [
        rowblk,
        rowblk,
        pl.BlockSpec((1, 1, _R), lambda i: (i, 0, 0)),
        vec, mat, vec, vec, vec, mat, vec,
    ]
    out_shape = [jax.ShapeDtypeStruct((2 * _B, _D), jnp.float32)]
    out_specs = [pl.BlockSpec((2 * _B, _D), lambda i: (0, 0))]
    if has_next:
        in_specs.append(mat)
        out_shape.append(jax.ShapeDtypeStruct((2 * _NP, _D), jnp.float32))
        out_specs.append(rowblk)
    return pl.pallas_call(
        body, grid=(_G,), in_specs=in_specs,
        out_specs=out_specs, out_shape=out_shape,
    )


_layer_mid = _make_layer(True)
_layer_last = _make_layer(False)


_F3 = 3 * _D
_TN = 16
_H = _F3 // 2


def _head_body(p0_ref, p1_ref, p2_ref,
               ow0_ref, ob0_ref, ow1_ref, ob1_ref, ow2_ref, ob2_ref,
               wt3_ref, wbt_ref, nb_ref,
               cw1_ref, cb1_ref, cw2_ref, cb2_ref,
               sw1_ref, sb1_ref, sw2_ref, sb2_ref,
               tw1_ref, tb1_ref, tw2_ref, tb2_ref,
               o_ref):
    prs = []
    for p_ref, ow_ref, ob_ref in ((p0_ref, ow0_ref, ob0_ref),
                                  (p1_ref, ow1_ref, ob1_ref),
                                  (p2_ref, ow2_ref, ob2_ref)):
        pr = jnp.dot(p_ref[...], ow_ref[...],
                     preferred_element_type=jnp.float32) + ob_ref[...]
        prs.append(jnp.maximum(pr, 0.0))
    qf = jnp.concatenate([pr[:_B] for pr in prs], axis=1)
    cf = jnp.concatenate([pr[_B:] for pr in prs], axis=1)

    diff = jnp.exp(-jnp.square(qf - cf))
    h = jnp.maximum(jnp.dot(diff, cw1_ref[...],
                            preferred_element_type=jnp.float32)
                    + cb1_ref[...], 0.0)
    sr = jnp.tanh(jnp.dot(h, cw2_ref[...],
                          preferred_element_type=jnp.float32) + cb2_ref[...])
    s = jnp.maximum(jnp.dot(sr, sw1_ref[...],
                            preferred_element_type=jnp.float32)
                    + sb1_ref[...], 0.0)
    score = jax.nn.sigmoid(jnp.dot(s, sw2_ref[...],
                                   preferred_element_type=jnp.float32)
                           + sb2_ref[...])

    cols = []
    for t in range(_TN):
        wt = wt3_ref[t * _F3:(t + 1) * _F3, :]
        tmp = jnp.dot(qf, wt, preferred_element_type=jnp.float32)
        cols.append(jnp.sum(tmp * cf, axis=1, keepdims=True))
    ntn_s = jnp.concatenate(cols, axis=1)
    blk = (jnp.dot(qf, wbt_ref[:_F3, :], preferred_element_type=jnp.float32)
           + jnp.dot(cf, wbt_ref[_F3:, :], preferred_element_type=jnp.float32))
    sim_rep = jnp.maximum(ntn_s + blk + nb_ref[...], 0.0)
    t1 = jnp.maximum(jnp.dot(sim_rep, tw1_ref[...],
                             preferred_element_type=jnp.float32)
                     + tb1_ref[...], 0.0)
    sim = jax.nn.sigmoid(jnp.dot(t1, tw2_ref[...],
                                 preferred_element_type=jnp.float32)
                         + tb2_ref[...])
    o_ref[...] = jnp.concatenate([score, sim], axis=1)


_head = pl.pallas_call(
    _head_body,
    out_shape=jax.ShapeDtypeStruct((_B, 2), jnp.float32),
)


def kernel(query_x, query_edge_index, query_graph_idx, corpus_x,
           corpus_edge_index, corpus_graph_idx, batch_size, params):
    p = params
    pad = ((0, _NP - _N), (0, 0))
    X = jnp.concatenate([jnp.pad(query_x, pad), jnp.pad(corpus_x, pad)],
                        axis=0)
    src = jnp.concatenate([query_edge_index[0], corpus_edge_index[0] + _NP])
    dst = jnp.concatenate([query_edge_index[1], corpus_edge_index[1]])
    g3 = jnp.concatenate([
        jnp.pad(query_graph_idx, (0, _NP - _N), constant_values=2 * _B),
        jnp.pad(corpus_graph_idx + _B, (0, _NP - _N),
                constant_values=2 * _B)]).reshape(_G, 1, _R)
    zero_tile = jnp.zeros((_RPT, _D), jnp.float32)

    def v(a):
        return a.reshape(1, -1)

    z = _pre(X, p['gin_w1_0'])
    edge_agg = _get_edge_agg()
    pooled = []
    for i in range(3):
        aggz = edge_agg(z, src, dst, zero_tile)
        argv = [z, aggz, g3, v(p['gin_b1_%d' % i]), p['gin_w2_%d' % i],
                v(p['gin_b2_%d' % i]), v(p['bn_g_%d' % i]),
                v(p['bn_b_%d' % i]), p['inner_w_%d' % i],
                v(p['inner_b_%d' % i])]
        if i < 2:
            pl_i, z = _layer_mid(*argv, p['gin_w1_%d' % (i + 1)])
        else:
            (pl_i,) = _layer_last(*argv)
        pooled.append(pl_i)

    wt3 = p['ntn_W'].transpose(2, 0, 1).reshape(_TN * _F3, _F3)
    o = _head(pooled[0], pooled[1], pooled[2],
              p['outer_w_0'], v(p['outer_b_0']),
              p['outer_w_1'], v(p['outer_b_1']),
              p['outer_w_2'], v(p['outer_b_2']),
              wt3, p['ntn_Wb'].T, v(p['ntn_bias']),
              p['cs_w1'], v(p['cs_b1']), p['cs_w2'], v(p['cs_b2']),
              p['sl_w1'], v(p['sl_b1']), p['sl_w2'], v(p['sl_b2']),
              p['ssl_w1'], v(p['ssl_b1']), p['ssl_w2'], v(p['ssl_b2']))
    return p['alpha'] * o[:, 0] + p['beta'] * o[:, 1]


# speedup vs baseline: 8.5693x; 2.7279x over previous
import functools

import jax
import jax.numpy as jnp
from jax import lax
from jax.experimental import pallas as pl
from jax.experimental.pallas import tpu as pltpu
from jax.experimental.pallas import tpu_sc as plsc

_N = 10000
_NP = 10240
_E = 320000
_B = 128
_D = 64
_IN = 128
_K = 80
_TILES = 16
_EPT = _E // _TILES
_CH = _EPT // _K
_RPT = _NP // _TILES
_R = 2048
_G = (2 * _NP) // _R


_NB = 10
_NI = _CH // _NB


def _edge_agg_body(z_hbm, src2_hbm, dst2_hbm, zero_hbm, out_hbm,
                   srcs_v, dsts_v, rows_v, agg_sh, sem_i, sems_g, sems_s):
    c = lax.axis_index("c")
    s = lax.axis_index("s")
    pltpu.sync_copy(zero_hbm, agg_sh.at[pl.ds(s * _RPT, _RPT)])
    plsc.subcore_barrier()
    crow0 = (c * _E + s * _EPT) // _K

    def body(i, carry):
        row0 = pl.multiple_of(crow0 + i * _NB, _NB)
        ci = pltpu.async_copy(src2_hbm.at[pl.ds(row0, _NB)], srcs_v, sem_i)
        cd = pltpu.async_copy(dst2_hbm.at[pl.ds(row0, _NB)], dsts_v, sem_i)
        ci.wait()
        cd.wait()
        gathers = [
            pltpu.async_copy(z_hbm.at[srcs_v.at[b]], rows_v.at[b],
                             sems_g.at[b])
            for b in range(_NB)
        ]
        scatters = []
        for b in range(_NB):
            gathers[b].wait()
            scatters.append(
                pltpu.async_copy(rows_v.at[b], agg_sh.at[dsts_v.at[b]],
                                 sems_s.at[b], add=True))
        for b in range(_NB):
            scatters[b].wait()
        return carry

    lax.fori_loop(0, _NI, body, 0)
    plsc.subcore_barrier()
    rbase = s * _RPT
    pltpu.sync_copy(agg_sh.at[pl.ds(rbase, _RPT)],
                    out_hbm.at[pl.ds(c * _NP + rbase, _RPT)])


@functools.cache
def _get_edge_agg():
    return pl.kernel(
        _edge_agg_body,
        out_type=jax.ShapeDtypeStruct((2 * _NP, _D), jnp.float32),
        mesh=plsc.VectorSubcoreMesh(core_axis_name="c", subcore_axis_name="s"),
        scratch_types=[
            pltpu.VMEM((_NB, _K), jnp.int32),
            pltpu.VMEM((_NB, _K), jnp.int32),
            pltpu.VMEM((_NB, _K, _D), jnp.float32),
            pltpu.VMEM_SHARED((_NP, _D), jnp.float32),
            pltpu.SemaphoreType.DMA,
            pltpu.SemaphoreType.DMA((_NB,)),
            pltpu.SemaphoreType.DMA((_NB,)),
        ],
        compiler_params=pltpu.CompilerParams(use_tc_tiling_on_sc=False),
    )


def _pre_body(x_ref, w_ref, o_ref):
    o_ref[...] = jnp.dot(x_ref[...], w_ref[...],
                         preferred_element_type=jnp.float32)


_pre = pl.pallas_call(
    _pre_body,
    grid=(_G,),
    in_specs=[
        pl.BlockSpec((_R, _IN), lambda i: (i, 0)),
        pl.BlockSpec((_IN, _D), lambda i: (0, 0)),
    ],
    out_specs=pl.BlockSpec((_R, _D), lambda i: (i, 0)),
    out_shape=jax.ShapeDtypeStruct((2 * _NP, _D), jnp.float32),
)


def _layer_body(has_next):
    def body(z_ref, a_ref, g_ref, b1_ref, w2_ref, b2_ref, bng_ref, bnb_ref,
             iw_ref, ib_ref, *rest):
        if has_next:
            w1n_ref, pooled_ref, znext_ref = rest
        else:
            (pooled_ref,) = rest
        i = pl.program_id(0)
        h = jnp.maximum(z_ref[...] + a_ref[...] + b1_ref[...], 0.0)
        h2 = jnp.dot(h, w2_ref[...], preferred_element_type=jnp.float32)
        h2 = h2 + b2_ref[...]
        x_new = jnp.maximum(h2 * bng_ref[...] + bnb_ref[...], 0.0)
        y = jnp.dot(x_new, iw_ref[...], preferred_element_type=jnp.float32)
        y = jnp.maximum(y + ib_ref[...], 0.0)
        g = g_ref[0, 0, :]
        oh = (g[None, :] == lax.broadcasted_iota(jnp.int32, (2 * _B, _R), 0))
        ps = jnp.dot(oh.astype(jnp.float32), y,
                     preferred_element_type=jnp.float32)

        @pl.when(i == 0)
        def _():
            pooled_ref[...] = ps

        @pl.when(i > 0)
        def _():
            pooled_ref[...] += ps

        if has_next:
            znext_ref[...] = jnp.dot(x_new, w1n_ref[...],
                                     preferred_element_type=jnp.float32)

    return body


def _make_layer(has_next):
    body = _layer_body(has_next)
    vec = pl.BlockSpec((1, _D), lambda i: (0, 0))
    mat = pl.BlockSpec((_D, _D), lambda i: (0, 0))
    rowblk = pl.BlockSpec((_R, _D), lambda i: (i, 0))
    in_specs = [
        rowblk,
        rowblk,
        pl.BlockSpec((1, 1, _R), lambda i: (i, 0, 0)),
        vec, mat, vec, vec, vec, mat, vec,
    ]
    out_shape = [jax.ShapeDtypeStruct((2 * _B, _D), jnp.float32)]
    out_specs = [pl.BlockSpec((2 * _B, _D), lambda i: (0, 0))]
    if has_next:
        in_specs.append(mat)
        out_shape.append(jax.ShapeDtypeStruct((2 * _NP, _D), jnp.float32))
        out_specs.append(rowblk)
    return pl.pallas_call(
        body, grid=(_G,), in_specs=in_specs,
        out_specs=out_specs, out_shape=out_shape,
    )


_layer_mid = _make_layer(True)
_layer_last = _make_layer(False)


_F3 = 3 * _D
_TN = 16
_H = _F3 // 2


def _head_body(p0_ref, p1_ref, p2_ref,
               ow0_ref, ob0_ref, ow1_ref, ob1_ref, ow2_ref, ob2_ref,
               wt3_ref, wbt_ref, nb_ref,
               cw1_ref, cb1_ref, cw2_ref, cb2_ref,
               sw1_ref, sb1_ref, sw2_ref, sb2_ref,
               tw1_ref, tb1_ref, tw2_ref, tb2_ref,
               o_ref):
    prs = []
    for p_ref, ow_ref, ob_ref in ((p0_ref, ow0_ref, ob0_ref),
                                  (p1_ref, ow1_ref, ob1_ref),
                                  (p2_ref, ow2_ref, ob2_ref)):
        pr = jnp.dot(p_ref[...], ow_ref[...],
                     preferred_element_type=jnp.float32) + ob_ref[...]
        prs.append(jnp.maximum(pr, 0.0))
    qf = jnp.concatenate([pr[:_B] for pr in prs], axis=1)
    cf = jnp.concatenate([pr[_B:] for pr in prs], axis=1)

    diff = jnp.exp(-jnp.square(qf - cf))
    h = jnp.maximum(jnp.dot(diff, cw1_ref[...],
                            preferred_element_type=jnp.float32)
                    + cb1_ref[...], 0.0)
    sr = jnp.tanh(jnp.dot(h, cw2_ref[...],
                          preferred_element_type=jnp.float32) + cb2_ref[...])
    s = jnp.maximum(jnp.dot(sr, sw1_ref[...],
                            preferred_element_type=jnp.float32)
                    + sb1_ref[...], 0.0)
    score = jax.nn.sigmoid(jnp.dot(s, sw2_ref[...],
                                   preferred_element_type=jnp.float32)
                           + sb2_ref[...])

    cols = []
    for t in range(_TN):
        wt = wt3_ref[t * _F3:(t + 1) * _F3, :]
        tmp = jnp.dot(qf, wt, preferred_element_type=jnp.float32)
        cols.append(jnp.sum(tmp * cf, axis=1, keepdims=True))
    ntn_s = jnp.concatenate(cols, axis=1)
    blk = (jnp.dot(qf, wbt_ref[:_F3, :], preferred_element_type=jnp.float32)
           + jnp.dot(cf, wbt_ref[_F3:, :], preferred_element_type=jnp.float32))
    sim_rep = jnp.maximum(ntn_s + blk + nb_ref[...], 0.0)
    t1 = jnp.maximum(jnp.dot(sim_rep, tw1_ref[...],
                             preferred_element_type=jnp.float32)
                     + tb1_ref[...], 0.0)
    sim = jax.nn.sigmoid(jnp.dot(t1, tw2_ref[...],
                                 preferred_element_type=jnp.float32)
                         + tb2_ref[...])
    o_ref[...] = jnp.concatenate([score, sim], axis=1)


_head = pl.pallas_call(
    _head_body,
    out_shape=jax.ShapeDtypeStruct((_B, 2), jnp.float32),
)


def kernel(query_x, query_edge_index, query_graph_idx, corpus_x,
           corpus_edge_index, corpus_graph_idx, batch_size, params):
    p = params
    pad = ((0, _NP - _N), (0, 0))
    X = jnp.concatenate([jnp.pad(query_x, pad), jnp.pad(corpus_x, pad)],
                        axis=0)
    src = jnp.concatenate([query_edge_index[0],
                           corpus_edge_index[0] + _NP]).reshape(-1, _K)
    dst = jnp.concatenate([query_edge_index[1],
                           corpus_edge_index[1]]).reshape(-1, _K)
    g3 = jnp.concatenate([
        jnp.pad(query_graph_idx, (0, _NP - _N), constant_values=2 * _B),
        jnp.pad(corpus_graph_idx + _B, (0, _NP - _N),
                constant_values=2 * _B)]).reshape(_G, 1, _R)
    zero_tile = jnp.zeros((_RPT, _D), jnp.float32)

    def v(a):
        return a.reshape(1, -1)

    z = _pre(X, p['gin_w1_0'])
    edge_agg = _get_edge_agg()
    pooled = []
    for i in range(3):
        aggz = edge_agg(z, src, dst, zero_tile)
        argv = [z, aggz, g3, v(p['gin_b1_%d' % i]), p['gin_w2_%d' % i],
                v(p['gin_b2_%d' % i]), v(p['bn_g_%d' % i]),
                v(p['bn_b_%d' % i]), p['inner_w_%d' % i],
                v(p['inner_b_%d' % i])]
        if i < 2:
            pl_i, z = _layer_mid(*argv, p['gin_w1_%d' % (i + 1)])
        else:
            (pl_i,) = _layer_last(*argv)
        pooled.append(pl_i)

    wt3 = p['ntn_W'].transpose(2, 0, 1).reshape(_TN * _F3, _F3)
    o = _head(pooled[0], pooled[1], pooled[2],
              p['outer_w_0'], v(p['outer_b_0']),
              p['outer_w_1'], v(p['outer_b_1']),
              p['outer_w_2'], v(p['outer_b_2']),
              wt3, p['ntn_Wb'].T, v(p['ntn_bias']),
              p['cs_w1'], v(p['cs_b1']), p['cs_w2'], v(p['cs_b2']),
              p['sl_w1'], v(p['sl_b1']), p['sl_w2'], v(p['sl_b2']),
              p['ssl_w1'], v(p['ssl_b1']), p['ssl_w2'], v(p['ssl_b2']))
    return p['alpha'] * o[:, 0] + p['beta'] * o[:, 1]
